# Initial kernel scaffold; baseline (speedup 1.0000x reference)
import jax
import jax.numpy as jnp
from jax.experimental import pallas as pl


def kernel(card_indices, table):
    raise NotImplementedError("write your pallas kernel here")


# speedup vs baseline: 3.9466x; 3.9466x over previous
import functools

import jax
import jax.numpy as jnp
from jax import lax
from jax.experimental import pallas as pl
from jax.experimental.pallas import tpu as pltpu
from jax.experimental.pallas import tpu_sc as plsc

_CHUNK = 512


@functools.cache
def _make_gather(B, V, D, chunk):
    info = plsc.get_sparse_core_info()
    num_workers = info.num_cores * info.num_subcores
    b_per_w = B // num_workers
    n_chunks = b_per_w // chunk
    assert b_per_w * num_workers == B and n_chunks * chunk == b_per_w

    mesh = plsc.VectorSubcoreMesh(core_axis_name="c", subcore_axis_name="s")

    @functools.partial(
        pl.kernel,
        mesh=mesh,
        out_type=jax.ShapeDtypeStruct((B, D), jnp.float32),
        scratch_types=[
            pltpu.VMEM((chunk,), jnp.int32),
            pltpu.VMEM((chunk, D), jnp.float32),
            pltpu.SemaphoreType.DMA,
        ],
        compiler_params=pltpu.CompilerParams(use_tc_tiling_on_sc=False),
    )
    def gather_kernel(idx_hbm, table_hbm, out_hbm, idx_v, rows_v, sem):
        wid = lax.axis_index("s") * info.num_cores + lax.axis_index("c")
        base = wid * b_per_w

        def body(c, carry):
            start = base + c * chunk
            pltpu.sync_copy(idx_hbm.at[pl.ds(start, chunk)], idx_v)
            pltpu.async_copy(table_hbm.at[idx_v], rows_v, sem).wait()
            pltpu.sync_copy(rows_v, out_hbm.at[pl.ds(start, chunk)])
            return carry

        lax.fori_loop(0, n_chunks, body, 0)

    return gather_kernel


def kernel(card_indices, table):
    batch, seq = card_indices.shape
    vocab, dim = table.shape
    idx_flat = card_indices.reshape(-1).astype(jnp.int32)
    gather = _make_gather(batch * seq, vocab, dim, _CHUNK)
    out = gather(idx_flat, table)
    return out.reshape(batch, seq, dim)


# speedup vs baseline: 4.2702x; 1.0820x over previous
import functools

import jax
import jax.numpy as jnp
from jax import lax
from jax.experimental import pallas as pl
from jax.experimental.pallas import tpu as pltpu
from jax.experimental.pallas import tpu_sc as plsc

_CHUNK = 512
_NBUF = 2


@functools.cache
def _make_gather(B, V, D, chunk, nbuf):
    info = plsc.get_sparse_core_info()
    num_workers = info.num_cores * info.num_subcores
    b_per_w = B // num_workers
    n_chunks = b_per_w // chunk
    n_groups = n_chunks // nbuf
    assert b_per_w * num_workers == B and n_groups * nbuf * chunk == b_per_w

    mesh = plsc.VectorSubcoreMesh(core_axis_name="c", subcore_axis_name="s")

    @functools.partial(
        pl.kernel,
        mesh=mesh,
        out_type=jax.ShapeDtypeStruct((B, D), jnp.float32),
        scratch_types=[
            pltpu.VMEM((b_per_w,), jnp.int32),
            *[pltpu.VMEM((chunk, D), jnp.float32) for _ in range(nbuf)],
            *[pltpu.SemaphoreType.DMA for _ in range(nbuf)],
            pltpu.SemaphoreType.DMA,
        ],
        compiler_params=pltpu.CompilerParams(use_tc_tiling_on_sc=False),
    )
    def gather_kernel(idx_hbm, table_hbm, out_hbm, idx_v, *bufs):
        rows = bufs[:nbuf]
        gsem = bufs[nbuf : 2 * nbuf]
        ssem = bufs[2 * nbuf]
        wid = lax.axis_index("s") * info.num_cores + lax.axis_index("c")
        base = wid * b_per_w

        pltpu.sync_copy(idx_hbm.at[pl.ds(base, b_per_w)], idx_v)

        def gather_start(b, c):
            pltpu.async_copy(
                table_hbm.at[idx_v.at[pl.ds(c * chunk, chunk)]], rows[b], gsem[b]
            )

        def gather_wait(b):
            pltpu.make_async_copy(
                table_hbm.at[idx_v.at[pl.ds(0, chunk)]], rows[b], gsem[b]
            ).wait()

        for b in range(nbuf):
            gather_start(b, b)

        def body(g, carry):
            for b in range(nbuf):
                c = g * nbuf + b
                gather_wait(b)
                copy = pltpu.make_async_copy(
                    rows[b], out_hbm.at[pl.ds(base + c * chunk, chunk)], ssem
                )
                copy.start()
                copy.wait()

                @pl.when(c + nbuf < n_chunks)
                def _():
                    gather_start(b, c + nbuf)

            return carry

        lax.fori_loop(0, n_groups, body, 0)

    return gather_kernel


def kernel(card_indices, table):
    batch, seq = card_indices.shape
    vocab, dim = table.shape
    idx_flat = card_indices.reshape(-1).astype(jnp.int32)
    gather = _make_gather(batch * seq, vocab, dim, _CHUNK, _NBUF)
    out = gather(idx_flat, table)
    return out.reshape(batch, seq, dim)
